# baseline (speedup 1.0000x reference)
import functools

import jax
import jax.numpy as jnp
from jax.experimental import pallas as pl
from jax.experimental.pallas import tpu as pltpu

_TILE = 16384


def _sums_kernel(d_hbm, s_hbm, m_hbm, out_ref, d_buf, s_buf, m_buf,
                 acc_l, acc_m, sem, *, rows, tile, minor):
    steps = rows // tile
    rem = rows - steps * tile
    hbms = (d_hbm.reshape(rows, minor), s_hbm.reshape(rows, minor),
            m_hbm.reshape(rows, minor))
    bufs = (d_buf, s_buf, m_buf)

    def _pieces(nrows):
        h0 = (nrows // 16) * 8
        return ((0, h0), (h0, nrows - h0)) if h0 else ((0, nrows),)

    def dma(slot, step, nrows):
        for k in range(3):
            for p, (off, cnt) in enumerate(_pieces(nrows)):
                pltpu.make_async_copy(
                    hbms[k].at[pl.ds(step * tile + off, cnt)],
                    bufs[k].at[slot, pl.ds(off, cnt)],
                    sem.at[k, slot, p]).start()

    def wait(slot, nrows):
        for k in range(3):
            for p, (off, cnt) in enumerate(_pieces(nrows)):
                pltpu.make_async_copy(
                    hbms[k].at[pl.ds(0, cnt)],
                    bufs[k].at[slot, pl.ds(off, cnt)],
                    sem.at[k, slot, p]).wait()

    def accumulate(d, s, m):
        l = (jnp.abs(d) + s) * m
        acc_l[...] += l.reshape(-1, 8, minor).sum(axis=0)
        acc_m[...] += m.reshape(-1, 8, minor).sum(axis=0)

    acc_l[...] = jnp.zeros_like(acc_l)
    acc_m[...] = jnp.zeros_like(acc_m)

    if steps > 0:
        dma(0, 0, tile)

        def body(j, _):
            cur = jax.lax.rem(j, 2)
            nxt = jax.lax.rem(j + 1, 2)

            @pl.when(j + 1 < steps)
            def _():
                dma(nxt, j + 1, tile)

            wait(cur, tile)
            accumulate(d_buf[cur], s_buf[cur], m_buf[cur])
            return ()

        jax.lax.fori_loop(0, steps, body, (), unroll=False)

    if rem:
        tslot = steps % 2
        dma(tslot, steps, rem)
        wait(tslot, rem)
        d = d_buf[tslot, :rem]
        s = s_buf[tslot, :rem]
        m = m_buf[tslot, :rem]
        pad = (-rem) % 8
        if pad:
            z = jnp.zeros((pad, minor), jnp.float32)
            d = jnp.concatenate([d, z], 0)
            s = jnp.concatenate([s, z], 0)
            m = jnp.concatenate([m, z], 0)
        accumulate(d, s, m)

    out_ref[0, 0] = acc_l[...].sum() / acc_m[...].sum()


def kernel(delta_norm, logstd, M_obs):
    f32 = jnp.float32
    if delta_norm.ndim < 2:
        delta_norm = delta_norm.reshape(1, -1)
        logstd = logstd.reshape(1, -1)
        M_obs = M_obs.reshape(1, -1)
    shape = delta_norm.shape
    minor = shape[-1]
    rows = delta_norm.size // minor
    tile = min(_TILE, max(8, (rows // 8) * 8))

    out = pl.pallas_call(
        functools.partial(_sums_kernel, rows=rows, tile=tile, minor=minor),
        out_shape=jax.ShapeDtypeStruct((1, 1), f32),
        in_specs=[pl.BlockSpec(memory_space=pltpu.MemorySpace.HBM)] * 3,
        out_specs=pl.BlockSpec(memory_space=pltpu.SMEM),
        scratch_shapes=[pltpu.VMEM((2, tile, minor), f32),
                        pltpu.VMEM((2, tile, minor), f32),
                        pltpu.VMEM((2, tile, minor), f32),
                        pltpu.VMEM((8, minor), f32),
                        pltpu.VMEM((8, minor), f32),
                        pltpu.SemaphoreType.DMA((3, 2, 2))],
        cost_estimate=pl.CostEstimate(
            flops=int(5 * delta_norm.size), transcendentals=0,
            bytes_accessed=int(12 * delta_norm.size)),
    )(delta_norm, logstd, M_obs)
    return out[0, 0]
